# baseline (speedup 1.0000x reference)
import functools

import jax
import jax.numpy as jnp
from jax import lax
from jax.experimental import pallas as pl
from jax.experimental.pallas import tpu as pltpu
from jax.experimental.pallas import tpu_sc as plsc

B = 16384
D = 64
NC = 2
NS = 16
NW = NC * NS
BPW = B // NW
TPB = 2048


def _tp_body(src, dst):
    s = src[...]
    dst[:, :D] = s[:, : TPB // 2].T
    dst[:, D:] = s[:, TPB // 2:].T


def _tc_transpose_pack(tbl_t, n_rows):
    nb = pl.cdiv(n_rows, TPB)
    return pl.pallas_call(
        _tp_body,
        grid=(nb,),
        in_specs=[pl.BlockSpec((D, TPB), lambda g: (0, g))],
        out_specs=pl.BlockSpec((TPB // 2, 2 * D), lambda g: (g, 0)),
        out_shape=jax.ShapeDtypeStruct((nb * TPB // 2, 2 * D), jnp.float32),
    )(tbl_t)


@functools.cache
def _make_sc_gather():
    mesh = plsc.VectorSubcoreMesh(
        core_axis_name="c", subcore_axis_name="s",
        num_cores=NC, num_subcores=NS)

    HP = BPW // 2

    @functools.partial(
        pl.kernel,
        out_type=jax.ShapeDtypeStruct((B, 2 * D), jnp.float32),
        mesh=mesh,
        scratch_types=[
            pltpu.VMEM((BPW,), jnp.int32),
            pltpu.VMEM((BPW,), jnp.int32),
            pltpu.VMEM((BPW,), jnp.int32),
            pltpu.VMEM((HP, 2 * D), jnp.float32),
            pltpu.VMEM((HP, 2 * D), jnp.float32),
            pltpu.SemaphoreType.DMA,
            pltpu.SemaphoreType.DMA,
        ],
    )
    def _sc_gather(u_hbm, i_hbm, upk_hbm, ipk_hbm, x_hbm,
                   uidx_v, iidx_v, oidx_v, st_v, x_v, sem_g, sem_o):
        wid = lax.axis_index("s") * NC + lax.axis_index("c")
        base = wid * BPW
        pltpu.sync_copy(u_hbm.at[pl.ds(base, BPW)], uidx_v)
        pltpu.sync_copy(i_hbm.at[pl.ds(base, BPW)], iidx_v)
        lanes = lax.iota(jnp.int32, 16)

        def fill_oidx(g, _):
            b16 = g * 16
            oidx_v[pl.ds(b16, 16)] = base + b16 + lanes
            return _

        lax.fori_loop(0, BPW // 16, fill_oidx, 0, unroll=False)

        for p in range(2):
            off = p * HP
            for idx_v, tbl, xoff in ((uidx_v, upk_hbm, 0),
                                     (iidx_v, ipk_hbm, D)):
                def issue(g, _, idx_v=idx_v, tbl=tbl):
                    vq = idx_v[pl.ds(off + g * 16, 16)]
                    vq = ((vq >> 11) << 10) + (vq & 1023)
                    for k in range(16):
                        pltpu.async_copy(
                            tbl.at[vq[k]], st_v.at[g * 16 + k], sem_g)
                    return _

                lax.fori_loop(0, HP // 16, issue, 0, unroll=False)
                pltpu.make_async_copy(
                    x_hbm.at[pl.ds(0, HP)], st_v, sem_g).wait()

                def select(g, _, idx_v=idx_v, xoff=xoff):
                    vh = ((idx_v[pl.ds(off + g * 16, 16)] >> 10) & 1) * D
                    for k in range(16):
                        hs = vh[k]
                        j = g * 16 + k
                        for t in range(D // 16):
                            x_v[j, pl.ds(xoff + t * 16, 16)] = (
                                st_v[j, pl.ds(hs + t * 16, 16)])
                    return _

                lax.fori_loop(0, HP // 16, select, 0, unroll=False)

            pltpu.async_copy(
                x_v, x_hbm.at[oidx_v.at[pl.ds(off, HP)]], sem_o).wait()

    return _sc_gather


BK = 2048


def _mlp_body(xg, ua, ia, Wu, Wi, bu, bi, W1, b1, W2, out):
    cdims = (((1,), (1,)), ((), ()))
    x = xg[...]
    u_e = x[:, :D] + lax.dot_general(ua[...], Wu[...], cdims,
                                     preferred_element_type=jnp.float32) + bu[...]
    i_e = x[:, D:] + lax.dot_general(ia[...], Wi[...], cdims,
                                     preferred_element_type=jnp.float32) + bi[...]
    w1 = W1[...]
    h = lax.dot_general(u_e, w1[:, :D], cdims,
                        preferred_element_type=jnp.float32)
    h = h + lax.dot_general(i_e, w1[:, D:], cdims,
                            preferred_element_type=jnp.float32)
    h = jnp.maximum(h + b1[...], 0.0)
    out[...] = lax.dot_general(h, W2[...], cdims,
                               preferred_element_type=jnp.float32)


def kernel(u, i, ua, ia, user_emb, item_emb, Wu, bu, Wi, bi, W1, b1, W2, b2):
    upk = _tc_transpose_pack(user_emb.T, 1000000)
    ipk = _tc_transpose_pack(item_emb.T, 100000)
    xg = _make_sc_gather()(u, i, upk, ipk)

    grid = (B // BK,)
    blk = lambda c: pl.BlockSpec((BK, c), lambda g: (g, 0))
    full = lambda shape: pl.BlockSpec(shape, lambda g: (0,) * len(shape))
    out = pl.pallas_call(
        _mlp_body,
        grid=grid,
        in_specs=[
            blk(2 * D),
            blk(128), blk(128),
            full((D, 128)), full((D, 128)),
            full((1, D)), full((1, D)),
            full((128, 128)), full((1, 128)),
            full((1, 128)),
        ],
        out_specs=pl.BlockSpec((BK, 1), lambda g: (g, 0)),
        out_shape=jax.ShapeDtypeStruct((B, 1), jnp.float32),
    )(xg, ua, ia, Wu, Wi,
      bu.reshape(1, D), bi.reshape(1, D), W1, b1.reshape(1, 128), W2)
    return out.reshape(B) + b2[0]


# speedup vs baseline: 1.5164x; 1.5164x over previous
import functools

import jax
import jax.numpy as jnp
from jax import lax
from jax.experimental import pallas as pl
from jax.experimental.pallas import tpu as pltpu
from jax.experimental.pallas import tpu_sc as plsc

B = 16384
D = 64
NC = 2
NS = 16
NW = NC * NS
BPW = B // NW
TPB = 4096
TSH = TPB.bit_length() - 1


def _make_tp_body(n_rows):
    def _tp_body(src, eye, dst):
        s = src[...]
        col = lax.broadcasted_iota(jnp.int32, (D, TPB), 1) + pl.program_id(0) * TPB
        s = jnp.where(col < n_rows, s, 0.0)
        a2 = jnp.concatenate([s[:, : TPB // 2], s[:, TPB // 2:]], axis=0)
        dst[...] = lax.dot_general(a2, eye[...], (((0,), (0,)), ((), ())),
                                   preferred_element_type=jnp.float32)
    return _tp_body


def _tc_transpose_pack(tbl_t, n_rows):
    nb = pl.cdiv(n_rows, TPB)
    eye = jnp.eye(2 * D, dtype=jnp.float32)
    return pl.pallas_call(
        _make_tp_body(n_rows),
        grid=(nb,),
        in_specs=[pl.BlockSpec((D, TPB), lambda g: (0, g)),
                  pl.BlockSpec((2 * D, 2 * D), lambda g: (0, 0))],
        out_specs=pl.BlockSpec((TPB // 2, 2 * D), lambda g: (g, 0)),
        out_shape=jax.ShapeDtypeStruct((nb * TPB // 2, 2 * D), jnp.float32),
    )(tbl_t, eye)


@functools.cache
def _make_sc_gather():
    mesh = plsc.VectorSubcoreMesh(
        core_axis_name="c", subcore_axis_name="s",
        num_cores=NC, num_subcores=NS)

    HP = BPW // 2

    @functools.partial(
        pl.kernel,
        out_type=jax.ShapeDtypeStruct((B, 2 * D), jnp.float32),
        mesh=mesh,
        scratch_types=[
            pltpu.VMEM((BPW,), jnp.int32),
            pltpu.VMEM((BPW,), jnp.int32),
            pltpu.VMEM((BPW,), jnp.int32),
            pltpu.VMEM((HP, 2 * D), jnp.float32),
            pltpu.VMEM((HP, 2 * D), jnp.float32),
            pltpu.SemaphoreType.DMA,
            pltpu.SemaphoreType.DMA,
        ],
    )
    def _sc_gather(u_hbm, i_hbm, upk_hbm, ipk_hbm, x_hbm,
                   uidx_v, iidx_v, oidx_v, st_v, x_v, sem_g, sem_o):
        wid = lax.axis_index("s") * NC + lax.axis_index("c")
        base = wid * BPW
        pltpu.sync_copy(u_hbm.at[pl.ds(base, BPW)], uidx_v)
        pltpu.sync_copy(i_hbm.at[pl.ds(base, BPW)], iidx_v)
        lanes = lax.iota(jnp.int32, 16)

        def fill_oidx(g, _):
            b16 = g * 16
            oidx_v[pl.ds(b16, 16)] = base + b16 + lanes
            return _

        lax.fori_loop(0, BPW // 16, fill_oidx, 0, unroll=False)

        for p in range(2):
            off = p * HP
            for idx_v, tbl, xoff in ((uidx_v, upk_hbm, 0),
                                     (iidx_v, ipk_hbm, D)):
                def issue(g, _, idx_v=idx_v, tbl=tbl):
                    vq = idx_v[pl.ds(off + g * 16, 16)]
                    vq = ((vq >> TSH) << (TSH - 1)) + (vq & (TPB // 2 - 1))
                    for k in range(16):
                        pltpu.async_copy(
                            tbl.at[vq[k]], st_v.at[g * 16 + k], sem_g)
                    return _

                lax.fori_loop(0, HP // 16, issue, 0, unroll=False)
                pltpu.make_async_copy(
                    x_hbm.at[pl.ds(0, HP)], st_v, sem_g).wait()

                def select(g, _, idx_v=idx_v, xoff=xoff):
                    vh = ((idx_v[pl.ds(off + g * 16, 16)] >> (TSH - 1)) & 1) * D
                    for k in range(16):
                        hs = vh[k]
                        j = g * 16 + k
                        for t in range(D // 16):
                            x_v[j, pl.ds(xoff + t * 16, 16)] = (
                                st_v[j, pl.ds(hs + t * 16, 16)])
                    return _

                lax.fori_loop(0, HP // 16, select, 0, unroll=False)

            pltpu.async_copy(
                x_v, x_hbm.at[oidx_v.at[pl.ds(off, HP)]], sem_o).wait()

    return _sc_gather


BK = 2048


def _mlp_body(xg, ua, ia, Wu, Wi, bu, bi, W1, b1, W2, out):
    cdims = (((1,), (1,)), ((), ()))
    x = xg[...]
    u_e = x[:, :D] + lax.dot_general(ua[...], Wu[...], cdims,
                                     preferred_element_type=jnp.float32) + bu[...]
    i_e = x[:, D:] + lax.dot_general(ia[...], Wi[...], cdims,
                                     preferred_element_type=jnp.float32) + bi[...]
    w1 = W1[...]
    h = lax.dot_general(u_e, w1[:, :D], cdims,
                        preferred_element_type=jnp.float32)
    h = h + lax.dot_general(i_e, w1[:, D:], cdims,
                            preferred_element_type=jnp.float32)
    h = jnp.maximum(h + b1[...], 0.0)
    out[...] = lax.dot_general(h, W2[...], cdims,
                               preferred_element_type=jnp.float32)


def kernel(u, i, ua, ia, user_emb, item_emb, Wu, bu, Wi, bi, W1, b1, W2, b2):
    upk = _tc_transpose_pack(user_emb.T, 1000000)
    ipk = _tc_transpose_pack(item_emb.T, 100000)
    xg = _make_sc_gather()(u, i, upk, ipk)

    grid = (B // BK,)
    blk = lambda c: pl.BlockSpec((BK, c), lambda g: (g, 0))
    full = lambda shape: pl.BlockSpec(shape, lambda g: (0,) * len(shape))
    out = pl.pallas_call(
        _mlp_body,
        grid=grid,
        in_specs=[
            blk(2 * D),
            blk(128), blk(128),
            full((D, 128)), full((D, 128)),
            full((1, D)), full((1, D)),
            full((128, 128)), full((1, 128)),
            full((1, 128)),
        ],
        out_specs=pl.BlockSpec((BK, 1), lambda g: (g, 0)),
        out_shape=jax.ShapeDtypeStruct((B, 1), jnp.float32),
    )(xg, ua, ia, Wu, Wi,
      bu.reshape(1, D), bi.reshape(1, D), W1, b1.reshape(1, 128), W2)
    return out.reshape(B) + b2[0]


# speedup vs baseline: 1.9656x; 1.2963x over previous
import functools

import jax
import jax.numpy as jnp
from jax import lax
from jax.experimental import pallas as pl
from jax.experimental.pallas import tpu as pltpu
from jax.experimental.pallas import tpu_sc as plsc

B = 16384
D = 64
NC = 2
NS = 16
NW = NC * NS
BPW = B // NW
TPB = 8192
TSH = TPB.bit_length() - 1


def _make_tp_body(n_rows):
    def _tp_body(src, eye, dst):
        s = src[...]
        col = lax.broadcasted_iota(jnp.int32, (D, TPB), 1) + pl.program_id(0) * TPB
        s = jnp.where(col < n_rows, s, 0.0)
        a2 = jnp.concatenate([s[:, : TPB // 2], s[:, TPB // 2:]], axis=0)
        dst[...] = lax.dot_general(a2, eye[...], (((0,), (0,)), ((), ())),
                                   preferred_element_type=jnp.float32)
    return _tp_body


def _tc_transpose_pack(tbl_t, n_rows):
    nb = pl.cdiv(n_rows, TPB)
    eye = jnp.eye(2 * D, dtype=jnp.float32)
    return pl.pallas_call(
        _make_tp_body(n_rows),
        grid=(nb,),
        in_specs=[pl.BlockSpec((D, TPB), lambda g: (0, g)),
                  pl.BlockSpec((2 * D, 2 * D), lambda g: (0, 0))],
        out_specs=pl.BlockSpec((TPB // 2, 2 * D), lambda g: (g, 0)),
        out_shape=jax.ShapeDtypeStruct((nb * TPB // 2, 2 * D), jnp.float32),
    )(tbl_t, eye)


@functools.cache
def _make_sc_gather():
    mesh = plsc.VectorSubcoreMesh(
        core_axis_name="c", subcore_axis_name="s",
        num_cores=NC, num_subcores=NS)

    HP = BPW // 2

    @functools.partial(
        pl.kernel,
        out_type=jax.ShapeDtypeStruct((B, 2 * D), jnp.float32),
        mesh=mesh,
        scratch_types=[
            pltpu.VMEM((BPW,), jnp.int32),
            pltpu.VMEM((BPW,), jnp.int32),
            pltpu.VMEM((BPW,), jnp.int32),
            pltpu.VMEM((HP, 2 * D), jnp.float32),
            pltpu.VMEM((HP, 2 * D), jnp.float32),
            pltpu.SemaphoreType.DMA,
            pltpu.SemaphoreType.DMA,
        ],
    )
    def _sc_gather(u_hbm, i_hbm, upk_hbm, ipk_hbm, x_hbm,
                   uidx_v, iidx_v, oidx_v, st_v, x_v, sem_g, sem_o):
        wid = lax.axis_index("s") * NC + lax.axis_index("c")
        base = wid * BPW
        pltpu.sync_copy(u_hbm.at[pl.ds(base, BPW)], uidx_v)
        pltpu.sync_copy(i_hbm.at[pl.ds(base, BPW)], iidx_v)
        lanes = lax.iota(jnp.int32, 16)

        def fill_oidx(g, _):
            b16 = g * 16
            oidx_v[pl.ds(b16, 16)] = base + b16 + lanes
            return _

        lax.fori_loop(0, BPW // 16, fill_oidx, 0, unroll=False)

        for p in range(2):
            off = p * HP
            for idx_v, tbl, xoff in ((uidx_v, upk_hbm, 0),
                                     (iidx_v, ipk_hbm, D)):
                def issue(g, _, idx_v=idx_v, tbl=tbl):
                    vq = idx_v[pl.ds(off + g * 16, 16)]
                    vq = ((vq >> TSH) << (TSH - 1)) + (vq & (TPB // 2 - 1))
                    for k in range(16):
                        pltpu.async_copy(
                            tbl.at[vq[k]], st_v.at[g * 16 + k], sem_g)
                    return _

                lax.fori_loop(0, HP // 16, issue, 0, unroll=False)
                pltpu.make_async_copy(
                    x_hbm.at[pl.ds(0, HP)], st_v, sem_g).wait()

                def select(g, _, idx_v=idx_v, xoff=xoff):
                    vh = ((idx_v[pl.ds(off + g * 16, 16)] >> (TSH - 1)) & 1) * D
                    for k in range(16):
                        hs = vh[k]
                        j = g * 16 + k
                        for t in range(D // 16):
                            x_v[j, pl.ds(xoff + t * 16, 16)] = (
                                st_v[j, pl.ds(hs + t * 16, 16)])
                    return _

                lax.fori_loop(0, HP // 16, select, 0, unroll=False)

            pltpu.async_copy(
                x_v, x_hbm.at[oidx_v.at[pl.ds(off, HP)]], sem_o).wait()

    return _sc_gather


BK = 2048


def _mlp_body(xg, ua, ia, Wu, Wi, bu, bi, W1, b1, W2, out):
    cdims = (((1,), (1,)), ((), ()))
    x = xg[...]
    u_e = x[:, :D] + lax.dot_general(ua[...], Wu[...], cdims,
                                     preferred_element_type=jnp.float32) + bu[...]
    i_e = x[:, D:] + lax.dot_general(ia[...], Wi[...], cdims,
                                     preferred_element_type=jnp.float32) + bi[...]
    w1 = W1[...]
    h = lax.dot_general(u_e, w1[:, :D], cdims,
                        preferred_element_type=jnp.float32)
    h = h + lax.dot_general(i_e, w1[:, D:], cdims,
                            preferred_element_type=jnp.float32)
    h = jnp.maximum(h + b1[...], 0.0)
    out[...] = lax.dot_general(h, W2[...], cdims,
                               preferred_element_type=jnp.float32)


def kernel(u, i, ua, ia, user_emb, item_emb, Wu, bu, Wi, bi, W1, b1, W2, b2):
    upk = _tc_transpose_pack(user_emb.T, 1000000)
    ipk = _tc_transpose_pack(item_emb.T, 100000)
    xg = _make_sc_gather()(u, i, upk, ipk)

    grid = (B // BK,)
    blk = lambda c: pl.BlockSpec((BK, c), lambda g: (g, 0))
    full = lambda shape: pl.BlockSpec(shape, lambda g: (0,) * len(shape))
    out = pl.pallas_call(
        _mlp_body,
        grid=grid,
        in_specs=[
            blk(2 * D),
            blk(128), blk(128),
            full((D, 128)), full((D, 128)),
            full((1, D)), full((1, D)),
            full((128, 128)), full((1, 128)),
            full((1, 128)),
        ],
        out_specs=pl.BlockSpec((BK, 1), lambda g: (g, 0)),
        out_shape=jax.ShapeDtypeStruct((B, 1), jnp.float32),
    )(xg, ua, ia, Wu, Wi,
      bu.reshape(1, D), bi.reshape(1, D), W1, b1.reshape(1, 128), W2)
    return out.reshape(B) + b2[0]
